# Initial kernel scaffold; baseline (speedup 1.0000x reference)
import jax
import jax.numpy as jnp
from jax.experimental import pallas as pl


def kernel(x, pos, batch, x_skip, pos_skip, batch_skip, W, b):
    raise NotImplementedError("write your pallas kernel here")


# speedup vs baseline: 10.7510x; 10.7510x over previous
import functools

import jax
import jax.numpy as jnp
from jax.experimental import pallas as pl

N_C = 4096
M_F = 16384
D_IN = 512
D_SKIP = 256
D_OUT = 512
BM = 128


def _body(posT_ref, batchf_ref, x_ref, ps_ref, bsf_ref, xs_ref,
          w1t_ref, w2t_ref, b_ref, out_ref):
    q0 = ps_ref[:, 0:1]
    q1 = ps_ref[:, 1:2]
    q2 = ps_ref[:, 2:3]
    p0 = posT_ref[0:1, :]
    p1 = posT_ref[1:2, :]
    p2 = posT_ref[2:3, :]
    d2 = (q0 - p0) ** 2 + (q1 - p1) ** 2 + (q2 - p2) ** 2
    d = jnp.sqrt(d2)
    same = bsf_ref[...] == batchf_ref[...]
    masked = jnp.where(same, d, jnp.inf)

    iota = jax.lax.broadcasted_iota(jnp.int32, (1, N_C), 1).astype(jnp.float32)
    a_mat = jnp.zeros((BM, N_C), jnp.float32)
    wsum = jnp.zeros((BM, 1), jnp.float32)
    for _ in range(3):
        mval = jnp.min(masked, axis=1, keepdims=True)
        tie = jnp.where(masked == mval, iota, float(N_C))
        midx = jnp.min(tie, axis=1, keepdims=True)
        w = 1.0 / (mval + 1e-8)
        hit = iota == midx
        a_mat = a_mat + jnp.where(hit, w, 0.0)
        wsum = wsum + w
        masked = jnp.where(hit, jnp.inf, masked)

    interp = jnp.dot(a_mat, x_ref[...],
                     preferred_element_type=jnp.float32) / (wsum + 1e-8)
    out_ref[...] = (jnp.dot(interp, w1t_ref[...],
                            preferred_element_type=jnp.float32)
                    + jnp.dot(xs_ref[...], w2t_ref[...],
                              preferred_element_type=jnp.float32)
                    + b_ref[...])


@functools.partial(jax.jit, static_argnames=("interpret",))
def kernel(x, pos, batch, x_skip, pos_skip, batch_skip, W, b,
           interpret=False):
    posT = pos.T
    batchf = batch.astype(jnp.float32).reshape(1, N_C)
    bsf = batch_skip.astype(jnp.float32).reshape(M_F, 1)
    WT = W.T
    w1t = WT[:D_IN]
    w2t = WT[D_IN:]
    b2 = b.reshape(1, D_OUT)

    grid = (M_F // BM,)
    y = pl.pallas_call(
        _body,
        grid=grid,
        in_specs=[
            pl.BlockSpec((3, N_C), lambda i: (0, 0)),
            pl.BlockSpec((1, N_C), lambda i: (0, 0)),
            pl.BlockSpec((N_C, D_IN), lambda i: (0, 0)),
            pl.BlockSpec((BM, 3), lambda i: (i, 0)),
            pl.BlockSpec((BM, 1), lambda i: (i, 0)),
            pl.BlockSpec((BM, D_SKIP), lambda i: (i, 0)),
            pl.BlockSpec((D_IN, D_OUT), lambda i: (0, 0)),
            pl.BlockSpec((D_SKIP, D_OUT), lambda i: (0, 0)),
            pl.BlockSpec((1, D_OUT), lambda i: (0, 0)),
        ],
        out_specs=pl.BlockSpec((BM, D_OUT), lambda i: (i, 0)),
        out_shape=jax.ShapeDtypeStruct((M_F, D_OUT), jnp.float32),
        interpret=interpret,
    )(posT, batchf, x, pos_skip, bsf, x_skip, w1t, w2t, b2)
    return y


# speedup vs baseline: 12.4278x; 1.1560x over previous
import functools

import jax
import jax.numpy as jnp
from jax.experimental import pallas as pl

N_C = 4096
M_F = 16384
D_IN = 512
D_SKIP = 256
D_OUT = 512
BM = 256


def _body(posT_ref, batchf_ref, x_ref, ps_ref, bsf_ref, xs_ref,
          w1t_ref, w2t_ref, b_ref, out_ref):
    q0 = ps_ref[:, 0:1]
    q1 = ps_ref[:, 1:2]
    q2 = ps_ref[:, 2:3]
    p0 = posT_ref[0:1, :]
    p1 = posT_ref[1:2, :]
    p2 = posT_ref[2:3, :]
    d2 = (q0 - p0) ** 2 + (q1 - p1) ** 2 + (q2 - p2) ** 2
    same = bsf_ref[...] == batchf_ref[...]
    masked = jnp.where(same, d2, jnp.inf)

    iota = jax.lax.broadcasted_iota(jnp.int32, (1, N_C), 1).astype(jnp.float32)
    a_mat = jnp.zeros((BM, N_C), jnp.bfloat16)
    wsum = jnp.zeros((BM, 1), jnp.float32)
    for _ in range(3):
        mval = jnp.min(masked, axis=1, keepdims=True)
        tie = jnp.where(masked == mval, iota, float(N_C))
        midx = jnp.min(tie, axis=1, keepdims=True)
        w = 1.0 / (jnp.sqrt(mval) + 1e-8)
        hit = iota == midx
        a_mat = a_mat + jnp.where(hit, w, 0.0).astype(jnp.bfloat16)
        wsum = wsum + w
        masked = jnp.where(hit, jnp.inf, masked)

    interp = jnp.dot(a_mat, x_ref[...].astype(jnp.bfloat16),
                     preferred_element_type=jnp.float32) / (wsum + 1e-8)
    out_ref[...] = (jnp.dot(interp.astype(jnp.bfloat16), w1t_ref[...],
                            preferred_element_type=jnp.float32)
                    + jnp.dot(xs_ref[...].astype(jnp.bfloat16), w2t_ref[...],
                              preferred_element_type=jnp.float32)
                    + b_ref[...])


@functools.partial(jax.jit, static_argnames=("interpret",))
def kernel(x, pos, batch, x_skip, pos_skip, batch_skip, W, b,
           interpret=False):
    posT = pos.T
    batchf = batch.astype(jnp.float32).reshape(1, N_C)
    bsf = batch_skip.astype(jnp.float32).reshape(M_F, 1)
    WT = W.T.astype(jnp.bfloat16)
    w1t = WT[:D_IN]
    w2t = WT[D_IN:]
    b2 = b.reshape(1, D_OUT)

    grid = (M_F // BM,)
    y = pl.pallas_call(
        _body,
        grid=grid,
        in_specs=[
            pl.BlockSpec((3, N_C), lambda i: (0, 0)),
            pl.BlockSpec((1, N_C), lambda i: (0, 0)),
            pl.BlockSpec((N_C, D_IN), lambda i: (0, 0)),
            pl.BlockSpec((BM, 3), lambda i: (i, 0)),
            pl.BlockSpec((BM, 1), lambda i: (i, 0)),
            pl.BlockSpec((BM, D_SKIP), lambda i: (i, 0)),
            pl.BlockSpec((D_IN, D_OUT), lambda i: (0, 0)),
            pl.BlockSpec((D_SKIP, D_OUT), lambda i: (0, 0)),
            pl.BlockSpec((1, D_OUT), lambda i: (0, 0)),
        ],
        out_specs=pl.BlockSpec((BM, D_OUT), lambda i: (i, 0)),
        out_shape=jax.ShapeDtypeStruct((M_F, D_OUT), jnp.float32),
        interpret=interpret,
    )(posT, batchf, x, pos_skip, bsf, x_skip, w1t, w2t, b2)
    return y
